# baseline (speedup 1.0000x reference)
import functools

import jax
import jax.numpy as jnp
from jax import lax
from jax.experimental import pallas as pl
from jax.experimental.pallas import tpu as pltpu
from jax.experimental.pallas import tpu_sc as plsc

_N_ENT = 1000000
_EMB = 64
_BATCH = 16384
_NC = 2
_NS = 16
_NW = _NC * _NS
_BPW = _BATCH // _NW
_CH = 128
_NCH = _BPW // _CH


def _sc_gather(ent_emb, rel_emb, heads, tails, relations, neg_heads, neg_tails):
    mesh = plsc.VectorSubcoreMesh(core_axis_name="c", subcore_axis_name="s")
    rows_t = jax.ShapeDtypeStruct((_BATCH, _EMB), jnp.float32)

    @functools.partial(
        pl.kernel,
        out_type=(rows_t,) * 5,
        mesh=mesh,
        scratch_types=(
            [pltpu.VMEM((_CH,), jnp.int32) for _ in range(5)]
            + [pltpu.VMEM((_CH, _EMB), jnp.float32) for _ in range(5)]
            + [pltpu.SemaphoreType.DMA]
        ),
        compiler_params=pltpu.CompilerParams(use_tc_tiling_on_sc=False),
    )
    def k(ent_hbm, rel_hbm, h_hbm, t_hbm, r_hbm, nh_hbm, nt_hbm,
          oh_hbm, ot_hbm, or_hbm, onh_hbm, ont_hbm,
          hi_v, ti_v, ri_v, nhi_v, nti_v,
          hr_v, tr_v, rr_v, nhr_v, ntr_v, sem):
        wid = lax.axis_index("s") * _NC + lax.axis_index("c")
        base = wid * _BPW
        ins = ((h_hbm, hi_v, ent_hbm, hr_v, oh_hbm),
               (t_hbm, ti_v, ent_hbm, tr_v, ot_hbm),
               (r_hbm, ri_v, rel_hbm, rr_v, or_hbm),
               (nh_hbm, nhi_v, ent_hbm, nhr_v, onh_hbm),
               (nt_hbm, nti_v, ent_hbm, ntr_v, ont_hbm))
        for c in range(_NCH):
            off = base + c * _CH
            for idx_hbm, idx_v, tab_hbm, rows_v, _ in ins:
                pltpu.sync_copy(idx_hbm.at[pl.ds(off, _CH)], idx_v)
            copies = [pltpu.async_copy(tab_hbm.at[idx_v], rows_v, sem)
                      for _, idx_v, tab_hbm, rows_v, _ in ins]
            for cp in copies:
                cp.wait()
            for _, _, _, rows_v, out_hbm in ins:
                pltpu.sync_copy(rows_v, out_hbm.at[pl.ds(off, _CH)])

    return k(ent_emb, rel_emb, heads, tails, relations, neg_heads, neg_tails)


def _tc_score(h, t, r, nh, nt):
    br = 2048

    def body(h_ref, t_ref, r_ref, nh_ref, nt_ref, pos_ref, neg_ref):
        rr = r_ref[...]
        d = h_ref[...] + rr - t_ref[...]
        pos_ref[...] = -jnp.sqrt(jnp.sum(d * d, axis=1))
        d = nh_ref[...] + rr - nt_ref[...]
        neg_ref[...] = -jnp.sqrt(jnp.sum(d * d, axis=1))

    return pl.pallas_call(
        body,
        grid=(_BATCH // br,),
        in_specs=[pl.BlockSpec((br, _EMB), lambda i: (i, 0))] * 5,
        out_specs=[pl.BlockSpec((br,), lambda i: (i,))] * 2,
        out_shape=[jax.ShapeDtypeStruct((_BATCH,), jnp.float32)] * 2,
    )(h, t, r, nh, nt)


def kernel(heads, tails, relations, negative_heads, negative_tails, ent_emb, rel_emb):
    idx = [x.astype(jnp.int32) for x in
           (heads, tails, relations, negative_heads, negative_tails)]
    rows = _sc_gather(ent_emb, rel_emb, *idx)
    pos, neg = _tc_score(*rows)
    return (pos, neg)


# speedup vs baseline: 1.0714x; 1.0714x over previous
import functools

import jax
import jax.numpy as jnp
from jax import lax
from jax.experimental import pallas as pl
from jax.experimental.pallas import tpu as pltpu
from jax.experimental.pallas import tpu_sc as plsc

_EMB = 64
_BATCH = 16384
_NC = 2
_NS = 16
_NW = _NC * _NS
_BPW = _BATCH // _NW
_CH = 128
_NCH = _BPW // _CH
_G = 16
_NG = _CH // _G


def _neg_sqrt(x):
    xc = jnp.maximum(x, jnp.float32(1e-30))
    i = plsc.bitcast(xc, jnp.int32)
    y = plsc.bitcast(jnp.int32(0x5F3759DF) - (i >> 1), jnp.float32)
    half = jnp.float32(0.5) * xc
    for _ in range(4):
        y = y * (jnp.float32(1.5) - half * y * y)
    return -(x * y)


def _score_chunk(hr, tr, rr, nhr, ntr, sp, sn, pos_v, neg_v):
    iota = lax.iota(jnp.int32, _G)

    @pl.loop(0, _NG)
    def _(g):
        row0 = g * _G
        for i in range(_G):
            row = row0 + i
            p = jnp.zeros((_G,), jnp.float32)
            pn = jnp.zeros((_G,), jnp.float32)
            for k in range(_EMB // _G):
                sl = pl.ds(k * _G, _G)
                rv = rr[row, sl]
                d = hr[row, sl] + rv - tr[row, sl]
                p = p + d * d
                dn = nhr[row, sl] + rv - ntr[row, sl]
                pn = pn + dn * dn
            sp[i, :] = p
            sn[i, :] = pn
        accp = jnp.zeros((_G,), jnp.float32)
        accn = jnp.zeros((_G,), jnp.float32)
        for j in range(_G):
            col = jnp.full((_G,), j, jnp.int32)
            accp = accp + plsc.load_gather(sp, [iota, col])
            accn = accn + plsc.load_gather(sn, [iota, col])
        pos_v[pl.ds(row0, _G)] = _neg_sqrt(accp)
        neg_v[pl.ds(row0, _G)] = _neg_sqrt(accn)


def _sc_score(ent_emb, rel_emb, heads, tails, relations, neg_heads, neg_tails):
    mesh = plsc.VectorSubcoreMesh(core_axis_name="c", subcore_axis_name="s")
    out_t = jax.ShapeDtypeStruct((_BATCH,), jnp.float32)

    @functools.partial(
        pl.kernel,
        out_type=(out_t, out_t),
        mesh=mesh,
        scratch_types=(
            [pltpu.VMEM((_CH,), jnp.int32) for _ in range(5)]
            + [pltpu.VMEM((_CH, _EMB), jnp.float32) for _ in range(5)]
            + [pltpu.VMEM((_G, _G), jnp.float32) for _ in range(2)]
            + [pltpu.VMEM((_CH,), jnp.float32) for _ in range(2)]
            + [pltpu.SemaphoreType.DMA]
        ),
        compiler_params=pltpu.CompilerParams(
            use_tc_tiling_on_sc=False, needs_layout_passes=False),
    )
    def k(ent_hbm, rel_hbm, h_hbm, t_hbm, r_hbm, nh_hbm, nt_hbm,
          pos_hbm, neg_hbm,
          hi_v, ti_v, ri_v, nhi_v, nti_v,
          hr_v, tr_v, rr_v, nhr_v, ntr_v,
          sp_v, sn_v, pos_v, neg_v, sem):
        wid = lax.axis_index("s") * _NC + lax.axis_index("c")
        base = wid * _BPW
        ins = ((h_hbm, hi_v, ent_hbm, hr_v),
               (t_hbm, ti_v, ent_hbm, tr_v),
               (r_hbm, ri_v, rel_hbm, rr_v),
               (nh_hbm, nhi_v, ent_hbm, nhr_v),
               (nt_hbm, nti_v, ent_hbm, ntr_v))
        for c in range(_NCH):
            off = base + c * _CH
            for idx_hbm, idx_v, _, _ in ins:
                pltpu.sync_copy(idx_hbm.at[pl.ds(off, _CH)], idx_v)
            copies = [pltpu.async_copy(tab_hbm.at[idx_v], rows_v, sem)
                      for _, idx_v, tab_hbm, rows_v in ins]
            for cp in copies:
                cp.wait()
            _score_chunk(hr_v, tr_v, rr_v, nhr_v, ntr_v, sp_v, sn_v,
                         pos_v, neg_v)
            pltpu.sync_copy(pos_v, pos_hbm.at[pl.ds(off, _CH)])
            pltpu.sync_copy(neg_v, neg_hbm.at[pl.ds(off, _CH)])

    return k(ent_emb, rel_emb, heads, tails, relations, neg_heads, neg_tails)


def kernel(heads, tails, relations, negative_heads, negative_tails, ent_emb, rel_emb):
    idx = [x.astype(jnp.int32) for x in
           (heads, tails, relations, negative_heads, negative_tails)]
    pos, neg = _sc_score(ent_emb, rel_emb, *idx)
    return (pos, neg)


# speedup vs baseline: 1.7140x; 1.5997x over previous
import functools

import jax
import jax.numpy as jnp
from jax import lax
from jax.experimental import pallas as pl
from jax.experimental.pallas import tpu as pltpu
from jax.experimental.pallas import tpu_sc as plsc

_EMB = 64
_BATCH = 16384
_NC = 2
_NS = 16
_NW = _NC * _NS
_BPW = _BATCH // _NW
_CH = 128
_NCH = _BPW // _CH
_G = 16
_NG = _CH // _G


def _neg_sqrt(x):
    xc = jnp.maximum(x, jnp.float32(1e-30))
    i = plsc.bitcast(xc, jnp.int32)
    y = plsc.bitcast(jnp.int32(0x5F3759DF) - (i >> 1), jnp.float32)
    half = jnp.float32(0.5) * xc
    for _ in range(4):
        y = y * (jnp.float32(1.5) - half * y * y)
    return -(x * y)


def _score_chunk(hr, tr, rr, nhr, ntr, sp, sn, pos_v, neg_v):
    iota = lax.iota(jnp.int32, _G)

    @pl.loop(0, _NG)
    def _(g):
        row0 = g * _G
        for i in range(_G):
            row = row0 + i
            p = jnp.zeros((_G,), jnp.float32)
            pn = jnp.zeros((_G,), jnp.float32)
            for k in range(_EMB // _G):
                sl = pl.ds(k * _G, _G)
                rv = rr[row, sl]
                d = hr[row, sl] + rv - tr[row, sl]
                p = p + d * d
                dn = nhr[row, sl] + rv - ntr[row, sl]
                pn = pn + dn * dn
            sp[i, :] = p
            sn[i, :] = pn
        accp = jnp.zeros((_G,), jnp.float32)
        accn = jnp.zeros((_G,), jnp.float32)
        for j in range(_G):
            col = jnp.full((_G,), j, jnp.int32)
            accp = accp + plsc.load_gather(sp, [iota, col])
            accn = accn + plsc.load_gather(sn, [iota, col])
        pos_v[pl.ds(row0, _G)] = _neg_sqrt(accp)
        neg_v[pl.ds(row0, _G)] = _neg_sqrt(accn)


def _sc_score(ent_emb, rel_emb, heads, tails, relations, neg_heads, neg_tails):
    mesh = plsc.VectorSubcoreMesh(core_axis_name="c", subcore_axis_name="s")
    out_t = jax.ShapeDtypeStruct((_BATCH,), jnp.float32)

    @functools.partial(
        pl.kernel,
        out_type=(out_t, out_t),
        mesh=mesh,
        scratch_types=(
            [pltpu.VMEM((_CH,), jnp.int32) for _ in range(5)]
            + [pltpu.VMEM((_CH, _EMB), jnp.float32) for _ in range(5)]
            + [pltpu.VMEM((_G, _G), jnp.float32) for _ in range(2)]
            + [pltpu.VMEM((_CH,), jnp.float32) for _ in range(2)]
            + [pltpu.SemaphoreType.DMA]
        ),
        compiler_params=pltpu.CompilerParams(needs_layout_passes=False),
    )
    def k(ent_hbm, rel_hbm, h_hbm, t_hbm, r_hbm, nh_hbm, nt_hbm,
          pos_hbm, neg_hbm,
          hi_v, ti_v, ri_v, nhi_v, nti_v,
          hr_v, tr_v, rr_v, nhr_v, ntr_v,
          sp_v, sn_v, pos_v, neg_v, sem):
        wid = lax.axis_index("s") * _NC + lax.axis_index("c")
        base = wid * _BPW
        ins = ((h_hbm, hi_v, ent_hbm, hr_v),
               (t_hbm, ti_v, ent_hbm, tr_v),
               (r_hbm, ri_v, rel_hbm, rr_v),
               (nh_hbm, nhi_v, ent_hbm, nhr_v),
               (nt_hbm, nti_v, ent_hbm, ntr_v))
        for c in range(_NCH):
            off = base + c * _CH
            for idx_hbm, idx_v, _, _ in ins:
                pltpu.sync_copy(idx_hbm.at[pl.ds(off, _CH)], idx_v)
            for _, idx_v, tab_hbm, rows_v in ins:
                def grp(j, idx_v=idx_v, tab_hbm=tab_hbm, rows_v=rows_v):
                    idx16 = idx_v[pl.ds(j * _G, _G)]
                    for i in range(_G):
                        e = idx16[i]
                        pltpu.async_copy(tab_hbm.at[e], rows_v.at[j * _G + i],
                                         sem)
                pl.loop(0, _CH // _G)(grp)
            for _, _, tab_hbm, rows_v in ins:
                pltpu.make_async_copy(
                    tab_hbm.at[pl.ds(0, _CH)], rows_v, sem).wait()
            _score_chunk(hr_v, tr_v, rr_v, nhr_v, ntr_v, sp_v, sn_v,
                         pos_v, neg_v)
            pltpu.sync_copy(pos_v, pos_hbm.at[pl.ds(off, _CH)])
            pltpu.sync_copy(neg_v, neg_hbm.at[pl.ds(off, _CH)])

    return k(ent_emb, rel_emb, heads, tails, relations, neg_heads, neg_tails)


def kernel(heads, tails, relations, negative_heads, negative_tails, ent_emb, rel_emb):
    idx = [x.astype(jnp.int32) for x in
           (heads, tails, relations, negative_heads, negative_tails)]
    pos, neg = _sc_score(ent_emb, rel_emb, *idx)
    return (pos, neg)
